# baseline (speedup 1.0000x reference)
import functools

import jax
import jax.numpy as jnp
from jax import lax
from jax.experimental import pallas as pl
from jax.experimental.pallas import tpu as pltpu
from jax.experimental.pallas import tpu_sc as plsc

BSZ, TSZ = 4, 2048
DIM = 1024
NUM_VARS = 320
GROUPS = 2
VAR_DIM = 128

N_ROWS = BSZ * TSZ
GV = GROUPS * NUM_VARS
ROW_TILE = 4096
N_TILES = N_ROWS // ROW_TILE
IDX_SUB = GROUPS * ROW_TILE // 128


def _tc_body(xa_ref, xb_ref, xc_ref, xd_ref, w_ref, b_ref, idx_ref, cnt_ref,
             ps_ref, cp_ref, pp_ref):
  i = pl.program_id(0)

  logits = jnp.concatenate(
      [jnp.dot(xa_ref[...], w_ref[...], preferred_element_type=jnp.float32),
       jnp.dot(xb_ref[...], w_ref[...], preferred_element_type=jnp.float32),
       jnp.dot(xc_ref[...], w_ref[...], preferred_element_type=jnp.float32),
       jnp.dot(xd_ref[...], w_ref[...], preferred_element_type=jnp.float32)],
      axis=0) + b_ref[...]

  col = lax.broadcasted_iota(jnp.int32, (ROW_TILE, GV), 1)
  in_g0 = col < NUM_VARS
  neg = jnp.float32(-jnp.inf)
  big = jnp.int32(GV)

  m0 = jnp.max(jnp.where(in_g0, logits, neg), axis=1, keepdims=True)
  m1 = jnp.max(jnp.where(in_g0, neg, logits), axis=1, keepdims=True)
  mb = jnp.where(in_g0, m0, m1)

  eq = logits == mb
  cw = jnp.where(eq, col, big)
  k0 = jnp.min(jnp.where(in_g0, cw, big), axis=1, keepdims=True)
  k1 = jnp.min(jnp.where(in_g0, big, cw), axis=1, keepdims=True)

  k0r = jnp.transpose(k0).reshape(ROW_TILE // 128, 128)
  k1r = jnp.transpose(k1).reshape(ROW_TILE // 128, 128)
  idx_ref[0] = jnp.concatenate([k0r, k1r], axis=0)

  e = jnp.exp(logits - mb)
  s0 = jnp.sum(jnp.where(in_g0, e, 0.0), axis=1, keepdims=True)
  s1 = jnp.sum(jnp.where(in_g0, 0.0, e), axis=1, keepdims=True)
  p = e * (1.0 / jnp.where(in_g0, s0, s1))
  ps_tile = jnp.sum(p, axis=0, keepdims=True)

  cnt_tile = jnp.sum(jnp.where(eq, 1.0, 0.0), axis=0, keepdims=True)

  @pl.when(i == 0)
  def _init():
    cnt_ref[...] = cnt_tile
    ps_ref[...] = ps_tile

  @pl.when(i > 0)
  def _acc():
    cnt_ref[...] += cnt_tile
    ps_ref[...] += ps_tile

  @pl.when(i == N_TILES - 1)
  def _finish():
    colf = lax.broadcasted_iota(jnp.int32, (1, GV), 1)
    g0 = colf < NUM_VARS
    inv_n = jnp.float32(1.0 / N_ROWS)

    hp = cnt_ref[...] * inv_n
    ent = hp * jnp.log(hp + 1e-7)
    ce0 = jnp.sum(jnp.where(g0, ent, 0.0), axis=1, keepdims=True)
    ce1 = jnp.sum(jnp.where(g0, 0.0, ent), axis=1, keepdims=True)
    cp_ref[...] = jnp.exp(-ce0) + jnp.exp(-ce1)

    ap = ps_ref[...] * inv_n
    pent = ap * jnp.log(ap + 1e-7)
    pe0 = jnp.sum(jnp.where(g0, pent, 0.0), axis=1, keepdims=True)
    pe1 = jnp.sum(jnp.where(g0, 0.0, pent), axis=1, keepdims=True)
    pp_ref[...] = jnp.exp(-pe0) + jnp.exp(-pe1)


def _tc_call(xf, W, b2):
  return pl.pallas_call(
      _tc_body,
      grid=(N_TILES,),
      in_specs=[
          pl.BlockSpec((ROW_TILE // 4, DIM), lambda i: (4 * i, 0)),
          pl.BlockSpec((ROW_TILE // 4, DIM), lambda i: (4 * i + 1, 0)),
          pl.BlockSpec((ROW_TILE // 4, DIM), lambda i: (4 * i + 2, 0)),
          pl.BlockSpec((ROW_TILE // 4, DIM), lambda i: (4 * i + 3, 0)),
          pl.BlockSpec((DIM, GV), lambda i: (0, 0)),
          pl.BlockSpec((1, GV), lambda i: (0, 0)),
      ],
      out_specs=[
          pl.BlockSpec((1, IDX_SUB, 128), lambda i: (i, 0, 0)),
          pl.BlockSpec((1, GV), lambda i: (0, 0)),
          pl.BlockSpec((1, GV), lambda i: (0, 0)),
          pl.BlockSpec((1, 1), lambda i: (0, 0)),
          pl.BlockSpec((1, 1), lambda i: (0, 0)),
      ],
      out_shape=[
          jax.ShapeDtypeStruct((N_TILES, IDX_SUB, 128), jnp.int32),
          jax.ShapeDtypeStruct((1, GV), jnp.float32),
          jax.ShapeDtypeStruct((1, GV), jnp.float32),
          jax.ShapeDtypeStruct((1, 1), jnp.float32),
          jax.ShapeDtypeStruct((1, 1), jnp.float32),
      ],
  )(xf, xf, xf, xf, W, b2)


_NW = 32
_BPW = GROUPS * N_ROWS // _NW
_CHUNK = 128
_NCH = _BPW // _CHUNK


def _sc_gather_body(table_hbm, idx_hbm, out_hbm, idx_v, rows_v, sem):
  wid = lax.axis_index("s") * 2 + lax.axis_index("c")
  ent0 = wid * _BPW
  tile = ent0 // (GROUPS * ROW_TILE)
  rem = ent0 % (GROUPS * ROW_TILE)
  grp = rem // ROW_TILE
  r0 = rem % ROW_TILE
  pltpu.sync_copy(idx_hbm.at[pl.ds((wid // 2) * 8, 2 * _NCH)], idx_v)
  sub = (wid % 2) * _NCH
  copies = []
  for j in range(_NCH):
    copies.append(
        pltpu.async_copy(table_hbm.at[idx_v.at[sub + j]],
                         rows_v.at[pl.ds(j * _CHUNK, _CHUNK)], sem))
  for c in copies:
    c.wait()
  row_start = pl.multiple_of(tile * ROW_TILE + r0, _BPW)
  pltpu.sync_copy(
      rows_v,
      out_hbm.at[pl.ds(row_start, _BPW), pl.ds(grp * VAR_DIM, VAR_DIM)])


@functools.lru_cache(maxsize=1)
def _make_sc_gather():
  return pl.kernel(
      _sc_gather_body,
      out_type=jax.ShapeDtypeStruct((N_ROWS, GROUPS * VAR_DIM), jnp.float32),
      mesh=plsc.VectorSubcoreMesh(core_axis_name="c", subcore_axis_name="s"),
      scratch_types=[
          pltpu.VMEM((2 * _NCH, _CHUNK), jnp.int32),
          pltpu.VMEM((_BPW, VAR_DIM), jnp.float32),
          pltpu.SemaphoreType.DMA,
      ],
  )


@jax.jit
def kernel(x, W, b, code_vars):
  xf = x.reshape(N_ROWS, DIM)
  b2 = b.reshape(1, GV)
  idx, _, _, cperp, pperp = _tc_call(xf, W, b2)

  table = code_vars.reshape(GV, VAR_DIM)
  out2d = _make_sc_gather()(table, idx.reshape(N_TILES * IDX_SUB, 128))
  out = out2d.reshape(BSZ, TSZ, GROUPS * VAR_DIM)
  return out, cperp[0, 0], pperp[0, 0]


# speedup vs baseline: 1.0592x; 1.0592x over previous
import functools

import jax
import jax.numpy as jnp
from jax import lax
from jax.experimental import pallas as pl
from jax.experimental.pallas import tpu as pltpu
from jax.experimental.pallas import tpu_sc as plsc

BSZ, TSZ = 4, 2048
DIM = 1024
NUM_VARS = 320
GROUPS = 2
VAR_DIM = 128

N_ROWS = BSZ * TSZ
GV = GROUPS * NUM_VARS
ROW_TILE = 2048
N_TILES = N_ROWS // ROW_TILE
IDX_SUB = GROUPS * ROW_TILE // 128


def _tc_body(xa_ref, xb_ref, xc_ref, xd_ref, w_ref, b_ref, idx_ref, cnt_ref,
             ps_ref, cp_ref, pp_ref):
  i = pl.program_id(0)

  logits = jnp.concatenate(
      [jnp.dot(xa_ref[...], w_ref[...], preferred_element_type=jnp.float32),
       jnp.dot(xb_ref[...], w_ref[...], preferred_element_type=jnp.float32),
       jnp.dot(xc_ref[...], w_ref[...], preferred_element_type=jnp.float32),
       jnp.dot(xd_ref[...], w_ref[...], preferred_element_type=jnp.float32)],
      axis=0) + b_ref[...][None, :]

  col = lax.broadcasted_iota(jnp.int32, (ROW_TILE, GV), 1)
  in_g0 = col < NUM_VARS
  neg = jnp.float32(-jnp.inf)
  big = jnp.int32(GV)

  m0 = jnp.max(jnp.where(in_g0, logits, neg), axis=1, keepdims=True)
  m1 = jnp.max(jnp.where(in_g0, neg, logits), axis=1, keepdims=True)
  mb = jnp.where(in_g0, m0, m1)

  eq = logits == mb
  cw = jnp.where(eq, col, big)
  k0 = jnp.min(jnp.where(in_g0, cw, big), axis=1, keepdims=True)
  k1 = jnp.min(jnp.where(in_g0, big, cw), axis=1, keepdims=True)

  k0r = jnp.transpose(k0).reshape(ROW_TILE // 128, 128)
  k1r = jnp.transpose(k1).reshape(ROW_TILE // 128, 128)
  idx_ref[0] = jnp.concatenate([k0r, k1r], axis=0)

  e = jnp.exp(logits - mb)
  s0 = jnp.sum(jnp.where(in_g0, e, 0.0), axis=1, keepdims=True)
  s1 = jnp.sum(jnp.where(in_g0, 0.0, e), axis=1, keepdims=True)
  p = e * (1.0 / jnp.where(in_g0, s0, s1))
  ps_tile = jnp.sum(p, axis=0, keepdims=True)

  cnt_tile = jnp.sum(jnp.where(eq, 1.0, 0.0), axis=0, keepdims=True)

  @pl.when(i == 0)
  def _init():
    cnt_ref[...] = cnt_tile
    ps_ref[...] = ps_tile

  @pl.when(i > 0)
  def _acc():
    cnt_ref[...] += cnt_tile
    ps_ref[...] += ps_tile

  @pl.when(i == N_TILES - 1)
  def _finish():
    colf = lax.broadcasted_iota(jnp.int32, (1, GV), 1)
    g0 = colf < NUM_VARS
    inv_n = jnp.float32(1.0 / N_ROWS)

    hp = cnt_ref[...] * inv_n
    ent = hp * jnp.log(hp + 1e-7)
    ce0 = jnp.sum(jnp.where(g0, ent, 0.0), axis=1, keepdims=True)
    ce1 = jnp.sum(jnp.where(g0, 0.0, ent), axis=1, keepdims=True)
    cp_ref[...] = jnp.exp(-ce0) + jnp.exp(-ce1)

    ap = ps_ref[...] * inv_n
    pent = ap * jnp.log(ap + 1e-7)
    pe0 = jnp.sum(jnp.where(g0, pent, 0.0), axis=1, keepdims=True)
    pe1 = jnp.sum(jnp.where(g0, 0.0, pent), axis=1, keepdims=True)
    pp_ref[...] = jnp.exp(-pe0) + jnp.exp(-pe1)


def _tc_call(xf, W, b2):
  return pl.pallas_call(
      _tc_body,
      grid=(N_TILES,),
      in_specs=[
          pl.BlockSpec((ROW_TILE // 4, DIM), lambda i: (4 * i, 0)),
          pl.BlockSpec((ROW_TILE // 4, DIM), lambda i: (4 * i + 1, 0)),
          pl.BlockSpec((ROW_TILE // 4, DIM), lambda i: (4 * i + 2, 0)),
          pl.BlockSpec((ROW_TILE // 4, DIM), lambda i: (4 * i + 3, 0)),
          pl.BlockSpec((DIM, GV), lambda i: (0, 0)),
          pl.BlockSpec((GV,), lambda i: (0,)),
      ],
      out_specs=[
          pl.BlockSpec((1, IDX_SUB, 128), lambda i: (i, 0, 0)),
          pl.BlockSpec((1, GV), lambda i: (0, 0)),
          pl.BlockSpec((1, GV), lambda i: (0, 0)),
          pl.BlockSpec((1, 1), lambda i: (0, 0)),
          pl.BlockSpec((1, 1), lambda i: (0, 0)),
      ],
      out_shape=[
          jax.ShapeDtypeStruct((N_TILES, IDX_SUB, 128), jnp.int32),
          jax.ShapeDtypeStruct((1, GV), jnp.float32),
          jax.ShapeDtypeStruct((1, GV), jnp.float32),
          jax.ShapeDtypeStruct((1, 1), jnp.float32),
          jax.ShapeDtypeStruct((1, 1), jnp.float32),
      ],
  )(xf, xf, xf, xf, W, b2)


_NW = 32
_BPW = GROUPS * N_ROWS // _NW
_CHUNK = 128
_NCH = _BPW // _CHUNK


def _sc_gather_body(table_hbm, idx_hbm, out_hbm, idx_v, rows_v, sem):
  wid = lax.axis_index("s") * 2 + lax.axis_index("c")
  ent0 = wid * _BPW
  tile = ent0 // (GROUPS * ROW_TILE)
  rem = ent0 % (GROUPS * ROW_TILE)
  grp = rem // ROW_TILE
  r0 = rem % ROW_TILE
  pltpu.sync_copy(idx_hbm.at[pl.ds((wid // 2) * 8, 2 * _NCH)], idx_v)
  sub = (wid % 2) * _NCH
  copies = []
  for j in range(_NCH):
    copies.append(
        pltpu.async_copy(table_hbm.at[idx_v.at[sub + j]],
                         rows_v.at[pl.ds(j * _CHUNK, _CHUNK)], sem))
  for c in copies:
    c.wait()
  row_start = pl.multiple_of(tile * ROW_TILE + r0, _BPW)
  pltpu.sync_copy(
      rows_v,
      out_hbm.at[pl.ds(row_start, _BPW), pl.ds(grp * VAR_DIM, VAR_DIM)])


@functools.lru_cache(maxsize=1)
def _make_sc_gather():
  return pl.kernel(
      _sc_gather_body,
      out_type=jax.ShapeDtypeStruct((N_ROWS, GROUPS * VAR_DIM), jnp.float32),
      mesh=plsc.VectorSubcoreMesh(core_axis_name="c", subcore_axis_name="s"),
      scratch_types=[
          pltpu.VMEM((2 * _NCH, _CHUNK), jnp.int32),
          pltpu.VMEM((_BPW, VAR_DIM), jnp.float32),
          pltpu.SemaphoreType.DMA,
      ],
  )


@jax.jit
def kernel(x, W, b, code_vars):
  xf = x.reshape(N_ROWS, DIM)
  b2 = b
  idx, _, _, cperp, pperp = _tc_call(xf, W, b2)

  table = code_vars.reshape(GV, VAR_DIM)
  out2d = _make_sc_gather()(table, idx.reshape(N_TILES * IDX_SUB, 128))
  out = out2d.reshape(BSZ, TSZ, GROUPS * VAR_DIM)
  return out, cperp[0, 0], pperp[0, 0]
